# baseline (speedup 1.0000x reference)
import functools

import jax
import jax.numpy as jnp
from jax import lax
from jax.experimental import pallas as pl
from jax.experimental.pallas import tpu as pltpu
from jax.experimental.pallas import tpu_sc as plsc

N_RBF = 20
CUTOFF = 5.0
FEAT = 128
OUTF = 3 * FEAT

N_NODES = 10000
N_EDGES = 320000

_PERM = tuple(3 * (c % 128) + (c // 128) for c in range(OUTF))

_NODE_BLK = 1000


def _to_bf16_bits(x):
    u = lax.bitcast_convert_type(x, jnp.uint32)
    lsb = (u >> 16) & jnp.uint32(1)
    return (u + jnp.uint32(0x7FFF) + lsb) >> 16


def _node_mlp_body(s_ref, w1_ref, b1_ref, w2_ref, b2_ref, o01_ref, o2_ref):
    h = jnp.dot(s_ref[...], w1_ref[...], preferred_element_type=jnp.float32)
    h = h + b1_ref[...]
    h = h * jax.nn.sigmoid(h)
    phi = jnp.dot(h, w2_ref[...], preferred_element_type=jnp.float32)
    phi = phi + b2_ref[...]
    r0 = _to_bf16_bits(phi[:, 0:128])
    r1 = _to_bf16_bits(phi[:, 128:256])
    o01_ref[...] = lax.bitcast_convert_type(r0 | (r1 << 16), jnp.int32)
    o2_ref[...] = phi[:, 256:384]


def _node_mlp(s_j, W1, b1, W2p, b2p):
    nblk = N_NODES // _NODE_BLK
    return pl.pallas_call(
        _node_mlp_body,
        grid=(nblk,),
        in_specs=[
            pl.BlockSpec((_NODE_BLK, FEAT), lambda i: (i, 0)),
            pl.BlockSpec((FEAT, FEAT), lambda i: (0, 0)),
            pl.BlockSpec((1, FEAT), lambda i: (0, 0)),
            pl.BlockSpec((FEAT, OUTF), lambda i: (0, 0)),
            pl.BlockSpec((1, OUTF), lambda i: (0, 0)),
        ],
        out_specs=[
            pl.BlockSpec((_NODE_BLK, FEAT), lambda i: (i, 0)),
            pl.BlockSpec((_NODE_BLK, FEAT), lambda i: (i, 0)),
        ],
        out_shape=[
            jax.ShapeDtypeStruct((N_NODES, FEAT), jnp.int32),
            jax.ShapeDtypeStruct((N_NODES, FEAT), jnp.float32),
        ],
    )(s_j, W1, b1.reshape(1, FEAT), W2p, b2p.reshape(1, OUTF))


_NC = 2
_NS = 16
_NW = _NC * _NS
_SLICES = (102400, 89600, 64000, 38400, 25600)
_SLICE_OFF = tuple(sum(_SLICES[:i]) for i in range(len(_SLICES) + 1))
_CHUNK = 200


def _sc_gather(t01, t2, idx, s):
    e_slice = _SLICES[s]
    e_base = _SLICE_OFF[s]
    e_per_w = e_slice // _NW
    nchunk = e_per_w // _CHUNK
    mesh = plsc.VectorSubcoreMesh(core_axis_name="c", subcore_axis_name="s")

    @functools.partial(
        pl.kernel,
        mesh=mesh,
        out_type=[
            jax.ShapeDtypeStruct((e_slice, FEAT), jnp.int32),
            jax.ShapeDtypeStruct((e_slice, FEAT), jnp.float32),
        ],
        scratch_types=[
            pltpu.VMEM((_CHUNK,), jnp.int32),
            pltpu.VMEM((_CHUNK,), jnp.int32),
            pltpu.VMEM((_CHUNK, FEAT), jnp.int32),
            pltpu.VMEM((_CHUNK, FEAT), jnp.int32),
            pltpu.VMEM((_CHUNK, FEAT), jnp.float32),
            pltpu.VMEM((_CHUNK, FEAT), jnp.float32),
            pltpu.SemaphoreType.DMA,
            pltpu.SemaphoreType.DMA,
        ],
    )
    def gather_kernel(
        t01_hbm, t2_hbm, idx_hbm, o01_hbm, o2_hbm,
        idx0_v, idx1_v, r01a_v, r01b_v, r2a_v, r2b_v, gsem, wsem,
    ):
        wid = lax.axis_index("s") * _NC + lax.axis_index("c")
        base = wid * e_per_w
        bufs = ((idx0_v, r01a_v, r2a_v), (idx1_v, r01b_v, r2b_v))
        npair = nchunk // 2

        def fire(c, b):
            idx_v, r01_v, r2_v = bufs[b]
            off = base + c * _CHUNK
            pltpu.sync_copy(
                idx_hbm.at[pl.ds(e_base + off, _CHUNK)], idx_v
            )
            pltpu.async_copy(t01_hbm.at[idx_v], r01_v, gsem)
            pltpu.async_copy(t2_hbm.at[idx_v], r2_v, gsem)

        def wait_gathers(b):
            idx_v, r01_v, r2_v = bufs[b]
            pltpu.make_async_copy(t01_hbm.at[idx_v], r01_v, gsem).wait()
            pltpu.make_async_copy(t2_hbm.at[idx_v], r2_v, gsem).wait()

        def writeback(c, b):
            _, r01_v, r2_v = bufs[b]
            off = base + c * _CHUNK
            pltpu.async_copy(r01_v, o01_hbm.at[pl.ds(off, _CHUNK)], wsem)
            pltpu.async_copy(r2_v, o2_hbm.at[pl.ds(off, _CHUNK)], wsem)

        def wait_writebacks(b):
            _, r01_v, r2_v = bufs[b]
            dummy = pl.ds(base, _CHUNK)
            pltpu.make_async_copy(r01_v, o01_hbm.at[dummy], wsem).wait()
            pltpu.make_async_copy(r2_v, o2_hbm.at[dummy], wsem).wait()

        fire(0, 0)

        def body(i, carry):
            c0 = 2 * i
            wait_gathers(0)
            writeback(c0, 0)

            @pl.when(i > 0)
            def _():
                wait_writebacks(1)

            fire(c0 + 1, 1)
            wait_gathers(1)
            writeback(c0 + 1, 1)

            @pl.when(i < npair - 1)
            def _():
                wait_writebacks(0)
                fire(c0 + 2, 0)

            return carry

        lax.fori_loop(0, npair, body, 0)
        wait_writebacks(0)
        wait_writebacks(1)

    return gather_kernel(t01, t2, idx)


_EDGE_BLK = 6400
_DROWS = _EDGE_BLK // FEAT

_S1 = 9.99999600e-01
_S3 = -1.66665526e-01
_S5 = 8.33240296e-03
_S7 = -1.98086326e-04
_S9 = 2.69971383e-06
_S11 = -2.03622121e-08
_TWO_PI = 6.283185307179586
_INV_TWO_PI = 0.15915494309189535


def _fast_sin(theta):
    k = jnp.round(theta * _INV_TWO_PI)
    r = theta - k * _TWO_PI
    r2 = r * r
    p = _S11
    p = p * r2 + _S9
    p = p * r2 + _S7
    p = p * r2 + _S5
    p = p * r2 + _S3
    p = p * r2 + _S1
    return p * r


def _mul_body(*refs):
    if len(refs) == 7:
        _, d_ref, g01_ref, g2_ref, wd_ref, bd_ref, out_ref = refs
    else:
        d_ref, g01_ref, g2_ref, wd_ref, bd_ref, out_ref = refs
    dt = d_ref[0].T
    d = jnp.concatenate([dt[:, r : r + 1] for r in range(_DROWS)], axis=0)
    n = lax.broadcasted_iota(jnp.int32, (1, N_RBF), 1).astype(jnp.float32) + 1.0
    coef = n * (jnp.pi / CUTOFF)
    num = _fast_sin(coef * d)
    denom = jnp.where(d == 0.0, 1.0, d)
    rbf = jnp.where(d == 0.0, 0.0, num / denom)
    w = jnp.dot(rbf, wd_ref[...], preferred_element_type=jnp.float32)
    w = w + bd_ref[...]
    u = lax.bitcast_convert_type(g01_ref[...], jnp.uint32)
    g0 = lax.bitcast_convert_type(u << 16, jnp.float32)
    g1 = lax.bitcast_convert_type(u & jnp.uint32(0xFFFF0000), jnp.float32)
    out_ref[...] = jnp.stack(
        [
            g0 * w[:, 0:128],
            g1 * w[:, 128:256],
            g2_ref[...] * w[:, 256:384],
        ],
        axis=0,
    )


def _mul_slice(prev, dist3, g01, g2, Wdp, bdp, s):
    nblk_s = _SLICES[s] // _EDGE_BLK
    blk0 = _SLICE_OFF[s] // _EDGE_BLK
    specs = [
        pl.BlockSpec((1, _DROWS, FEAT), lambda i: (i + blk0, 0, 0)),
        pl.BlockSpec((_EDGE_BLK, FEAT), lambda i: (i, 0)),
        pl.BlockSpec((_EDGE_BLK, FEAT), lambda i: (i, 0)),
        pl.BlockSpec((N_RBF, OUTF), lambda i: (0, 0)),
        pl.BlockSpec((1, OUTF), lambda i: (0, 0)),
    ]
    args = (dist3, g01, g2, Wdp, bdp)
    aliases = {}
    if prev is not None:
        specs = [pl.BlockSpec(memory_space=pl.ANY)] + specs
        args = (prev,) + args
        aliases = {0: 0}
    return pl.pallas_call(
        _mul_body,
        grid=(nblk_s,),
        in_specs=specs,
        out_specs=pl.BlockSpec(
            (3, _EDGE_BLK, FEAT), lambda i: (0, i + blk0, 0)
        ),
        out_shape=jax.ShapeDtypeStruct((3, N_EDGES, FEAT), jnp.float32),
        input_output_aliases=aliases,
    )(*args)


def kernel(s_j, dist, nbrs, W1, b1, W2, b2, Wd, bd):
    perm = jnp.asarray(_PERM, dtype=jnp.int32)
    W2p = W2[:, perm]
    b2p = b2[perm]
    Wdp = Wd[:, perm]
    bdp = bd[perm]
    t01, t2 = _node_mlp(s_j, W1, b1, W2p, b2p)
    idx = nbrs[:, 1].astype(jnp.int32)
    dist3 = dist.reshape(N_EDGES // _EDGE_BLK, _DROWS, FEAT)
    bdp2 = bdp.reshape(1, OUTF)
    nslice = len(_SLICES)
    gathered = [_sc_gather(t01, t2, idx, s) for s in range(nslice)]
    out = None
    for s in range(nslice):
        g01, g2 = gathered[s]
        out = _mul_slice(out, dist3, g01, g2, Wdp, bdp2, s)
    return out.transpose(1, 2, 0)


# speedup vs baseline: 1.0139x; 1.0139x over previous
import functools

import jax
import jax.numpy as jnp
from jax import lax
from jax.experimental import pallas as pl
from jax.experimental.pallas import tpu as pltpu
from jax.experimental.pallas import tpu_sc as plsc

N_RBF = 20
CUTOFF = 5.0
FEAT = 128
OUTF = 3 * FEAT

N_NODES = 10000
N_EDGES = 320000

_PERM = tuple(3 * (c % 128) + (c // 128) for c in range(OUTF))

_NODE_BLK = 1000


def _to_bf16_bits(x):
    u = lax.bitcast_convert_type(x, jnp.uint32)
    lsb = (u >> 16) & jnp.uint32(1)
    return (u + jnp.uint32(0x7FFF) + lsb) >> 16


def _node_mlp_body(s_ref, w1_ref, b1_ref, w2_ref, b2_ref, o01_ref, o2_ref):
    h = jnp.dot(s_ref[...], w1_ref[...], preferred_element_type=jnp.float32)
    h = h + b1_ref[...]
    h = h * jax.nn.sigmoid(h)
    phi = jnp.dot(h, w2_ref[...], preferred_element_type=jnp.float32)
    phi = phi + b2_ref[...]
    r0 = _to_bf16_bits(phi[:, 0:128])
    r1 = _to_bf16_bits(phi[:, 128:256])
    o01_ref[...] = lax.bitcast_convert_type(r0 | (r1 << 16), jnp.int32)
    o2_ref[...] = phi[:, 256:384]


def _node_mlp(s_j, W1, b1, W2p, b2p):
    nblk = N_NODES // _NODE_BLK
    return pl.pallas_call(
        _node_mlp_body,
        grid=(nblk,),
        in_specs=[
            pl.BlockSpec((_NODE_BLK, FEAT), lambda i: (i, 0)),
            pl.BlockSpec((FEAT, FEAT), lambda i: (0, 0)),
            pl.BlockSpec((1, FEAT), lambda i: (0, 0)),
            pl.BlockSpec((FEAT, OUTF), lambda i: (0, 0)),
            pl.BlockSpec((1, OUTF), lambda i: (0, 0)),
        ],
        out_specs=[
            pl.BlockSpec((_NODE_BLK, FEAT), lambda i: (i, 0)),
            pl.BlockSpec((_NODE_BLK, FEAT), lambda i: (i, 0)),
        ],
        out_shape=[
            jax.ShapeDtypeStruct((N_NODES, FEAT), jnp.int32),
            jax.ShapeDtypeStruct((N_NODES, FEAT), jnp.float32),
        ],
    )(s_j, W1, b1.reshape(1, FEAT), W2p, b2p.reshape(1, OUTF))


_NC = 2
_NS = 16
_NW = _NC * _NS
_SLICES = (64000, 64000, 64000, 64000, 64000)
_SLICE_OFF = tuple(sum(_SLICES[:i]) for i in range(len(_SLICES) + 1))
_CHUNK = 200


def _sc_gather(t01, t2, idx, s):
    e_slice = _SLICES[s]
    e_base = _SLICE_OFF[s]
    e_per_w = e_slice // _NW
    nchunk = e_per_w // _CHUNK
    mesh = plsc.VectorSubcoreMesh(core_axis_name="c", subcore_axis_name="s")

    @functools.partial(
        pl.kernel,
        mesh=mesh,
        out_type=[
            jax.ShapeDtypeStruct((e_slice, FEAT), jnp.int32),
            jax.ShapeDtypeStruct((e_slice, FEAT), jnp.float32),
        ],
        scratch_types=[
            pltpu.VMEM((_CHUNK,), jnp.int32),
            pltpu.VMEM((_CHUNK,), jnp.int32),
            pltpu.VMEM((_CHUNK, FEAT), jnp.int32),
            pltpu.VMEM((_CHUNK, FEAT), jnp.int32),
            pltpu.VMEM((_CHUNK, FEAT), jnp.float32),
            pltpu.VMEM((_CHUNK, FEAT), jnp.float32),
            pltpu.SemaphoreType.DMA,
            pltpu.SemaphoreType.DMA,
        ],
    )
    def gather_kernel(
        t01_hbm, t2_hbm, idx_hbm, o01_hbm, o2_hbm,
        idx0_v, idx1_v, r01a_v, r01b_v, r2a_v, r2b_v, gsem, wsem,
    ):
        wid = lax.axis_index("s") * _NC + lax.axis_index("c")
        base = wid * e_per_w
        bufs = ((idx0_v, r01a_v, r2a_v), (idx1_v, r01b_v, r2b_v))
        npair = nchunk // 2

        def fire(c, b):
            idx_v, r01_v, r2_v = bufs[b]
            off = base + c * _CHUNK
            pltpu.sync_copy(
                idx_hbm.at[pl.ds(e_base + off, _CHUNK)], idx_v
            )
            pltpu.async_copy(t01_hbm.at[idx_v], r01_v, gsem)
            pltpu.async_copy(t2_hbm.at[idx_v], r2_v, gsem)

        def wait_gathers(b):
            idx_v, r01_v, r2_v = bufs[b]
            pltpu.make_async_copy(t01_hbm.at[idx_v], r01_v, gsem).wait()
            pltpu.make_async_copy(t2_hbm.at[idx_v], r2_v, gsem).wait()

        def writeback(c, b):
            _, r01_v, r2_v = bufs[b]
            off = base + c * _CHUNK
            pltpu.async_copy(r01_v, o01_hbm.at[pl.ds(off, _CHUNK)], wsem)
            pltpu.async_copy(r2_v, o2_hbm.at[pl.ds(off, _CHUNK)], wsem)

        def wait_writebacks(b):
            _, r01_v, r2_v = bufs[b]
            dummy = pl.ds(base, _CHUNK)
            pltpu.make_async_copy(r01_v, o01_hbm.at[dummy], wsem).wait()
            pltpu.make_async_copy(r2_v, o2_hbm.at[dummy], wsem).wait()

        fire(0, 0)

        def body(i, carry):
            c0 = 2 * i
            wait_gathers(0)
            writeback(c0, 0)

            @pl.when(i > 0)
            def _():
                wait_writebacks(1)

            fire(c0 + 1, 1)
            wait_gathers(1)
            writeback(c0 + 1, 1)

            @pl.when(i < npair - 1)
            def _():
                wait_writebacks(0)
                fire(c0 + 2, 0)

            return carry

        lax.fori_loop(0, npair, body, 0)
        wait_writebacks(0)
        wait_writebacks(1)

    return gather_kernel(t01, t2, idx)


_EDGE_BLK = 6400
_DROWS = _EDGE_BLK // FEAT

_S1 = 9.99999600e-01
_S3 = -1.66665526e-01
_S5 = 8.33240296e-03
_S7 = -1.98086326e-04
_S9 = 2.69971383e-06
_S11 = -2.03622121e-08
_TWO_PI = 6.283185307179586
_INV_TWO_PI = 0.15915494309189535


def _fast_sin(theta):
    k = jnp.round(theta * _INV_TWO_PI)
    r = theta - k * _TWO_PI
    r2 = r * r
    p = _S11
    p = p * r2 + _S9
    p = p * r2 + _S7
    p = p * r2 + _S5
    p = p * r2 + _S3
    p = p * r2 + _S1
    return p * r


def _mul_body(*refs):
    if len(refs) == 7:
        _, d_ref, g01_ref, g2_ref, wd_ref, bd_ref, out_ref = refs
    else:
        d_ref, g01_ref, g2_ref, wd_ref, bd_ref, out_ref = refs
    dt = d_ref[0].T
    d = jnp.concatenate([dt[:, r : r + 1] for r in range(_DROWS)], axis=0)
    n = lax.broadcasted_iota(jnp.int32, (1, N_RBF), 1).astype(jnp.float32) + 1.0
    coef = n * (jnp.pi / CUTOFF)
    num = _fast_sin(coef * d)
    denom = jnp.where(d == 0.0, 1.0, d)
    rbf = jnp.where(d == 0.0, 0.0, num / denom)
    w = jnp.dot(rbf, wd_ref[...], preferred_element_type=jnp.float32)
    w = w + bd_ref[...]
    u = lax.bitcast_convert_type(g01_ref[...], jnp.uint32)
    g0 = lax.bitcast_convert_type(u << 16, jnp.float32)
    g1 = lax.bitcast_convert_type(u & jnp.uint32(0xFFFF0000), jnp.float32)
    out_ref[...] = jnp.stack(
        [
            g0 * w[:, 0:128],
            g1 * w[:, 128:256],
            g2_ref[...] * w[:, 256:384],
        ],
        axis=0,
    )


def _mul_slice(prev, dist3, g01, g2, Wdp, bdp, s):
    nblk_s = _SLICES[s] // _EDGE_BLK
    blk0 = _SLICE_OFF[s] // _EDGE_BLK
    specs = [
        pl.BlockSpec((1, _DROWS, FEAT), lambda i: (i + blk0, 0, 0)),
        pl.BlockSpec((_EDGE_BLK, FEAT), lambda i: (i, 0)),
        pl.BlockSpec((_EDGE_BLK, FEAT), lambda i: (i, 0)),
        pl.BlockSpec((N_RBF, OUTF), lambda i: (0, 0)),
        pl.BlockSpec((1, OUTF), lambda i: (0, 0)),
    ]
    args = (dist3, g01, g2, Wdp, bdp)
    aliases = {}
    if prev is not None:
        specs = [pl.BlockSpec(memory_space=pl.ANY)] + specs
        args = (prev,) + args
        aliases = {0: 0}
    return pl.pallas_call(
        _mul_body,
        grid=(nblk_s,),
        in_specs=specs,
        out_specs=pl.BlockSpec(
            (3, _EDGE_BLK, FEAT), lambda i: (0, i + blk0, 0)
        ),
        out_shape=jax.ShapeDtypeStruct((3, N_EDGES, FEAT), jnp.float32),
        input_output_aliases=aliases,
    )(*args)


def kernel(s_j, dist, nbrs, W1, b1, W2, b2, Wd, bd):
    perm = jnp.asarray(_PERM, dtype=jnp.int32)
    W2p = W2[:, perm]
    b2p = b2[perm]
    Wdp = Wd[:, perm]
    bdp = bd[perm]
    t01, t2 = _node_mlp(s_j, W1, b1, W2p, b2p)
    idx = nbrs[:, 1].astype(jnp.int32)
    dist3 = dist.reshape(N_EDGES // _EDGE_BLK, _DROWS, FEAT)
    bdp2 = bdp.reshape(1, OUTF)
    nslice = len(_SLICES)
    gathered = [_sc_gather(t01, t2, idx, s) for s in range(nslice)]
    out = None
    for s in range(nslice):
        g01, g2 = gathered[s]
        out = _mul_slice(out, dist3, g01, g2, Wdp, bdp2, s)
    return out.transpose(1, 2, 0)
